# baseline (speedup 1.0000x reference)
import functools

import jax
import jax.numpy as jnp
from jax import lax
from jax.experimental import pallas as pl
from jax.experimental.pallas import tpu as pltpu
from jax.experimental.pallas import tpu_sc as plsc

_DIMS = ([16, 32, 64] * 8) + [16, 32]
_NFIELD = 26
_VOCAB = 100000
_BATCH = 16384
_NC = 2
_NS = 16
_NW = _NC * _NS
_BPW = _BATCH // _NW
_CHUNK = 128
_NCHUNK = _BPW // _CHUNK
_VLEN = 16

_GROUPS = {
    16: [f for f in range(_NFIELD) if _DIMS[f] == 16],
    32: [f for f in range(_NFIELD) if _DIMS[f] == 32],
    64: [f for f in range(_NFIELD) if _DIMS[f] == 64],
}
_SCHED = [
    (f, d, i % 2)
    for d in (16, 32, 64)
    for i, f in enumerate(_GROUPS[d])
]
_OFFS = [0] * _NFIELD
for _d, _fs in _GROUPS.items():
    for _i, _f in enumerate(_fs):
        _OFFS[_f] = _i * _VOCAB


def _make_kernel():
    mesh = plsc.VectorSubcoreMesh(core_axis_name="c", subcore_axis_name="s")
    out_type = tuple(
        jax.ShapeDtypeStruct((_BATCH, _DIMS[f]), jnp.float32)
        for f in range(_NFIELD)
    )

    @functools.partial(
        pl.kernel,
        mesh=mesh,
        out_type=out_type,
        compiler_params=pltpu.CompilerParams(use_tc_tiling_on_sc=False),
        scratch_types=[
            pltpu.VMEM((_NFIELD * _NCHUNK, _CHUNK), jnp.int32),
            pltpu.VMEM((3, _NCHUNK, _CHUNK), jnp.int32),
            pltpu.VMEM((_BPW, 16), jnp.float32),
            pltpu.VMEM((_BPW, 16), jnp.float32),
            pltpu.VMEM((_BPW, 32), jnp.float32),
            pltpu.VMEM((_BPW, 32), jnp.float32),
            pltpu.VMEM((_BPW, 64), jnp.float32),
            pltpu.VMEM((_BPW, 64), jnp.float32),
            pltpu.SemaphoreType.DMA,
            pltpu.SemaphoreType.DMA,
            pltpu.SemaphoreType.DMA,
        ],
    )
    def run(xflat_hbm, sidx_hbm, s16, s32, s64, *rest):
        outs = rest[:_NFIELD]
        (sidx_v, idxs, b16a, b16b, b32a, b32b, b64a, b64b,
         gsem, esem, wsem) = rest[_NFIELD:]
        idxb = tuple(idxs.at[k] for k in range(3))
        dbuf = {16: (b16a, b16b), 32: (b32a, b32b), 64: (b64a, b64b)}
        stack = {16: s16, 32: s32, 64: s64}

        wid = lax.axis_index("s") * _NC + lax.axis_index("c")
        base = wid * _BPW
        pltpu.sync_copy(sidx_hbm.at[wid], sidx_v)

        def extract(step):
            f, _, _ = _SCHED[step]
            dst = idxb[step % 3]
            return [
                pltpu.async_copy(
                    xflat_hbm.at[sidx_v.at[f * _NCHUNK + c]],
                    dst.at[c],
                    esem,
                )
                for c in range(_NCHUNK)
            ]

        def fire(step):
            _, d, par = _SCHED[step]
            src_idx = idxb[step % 3]
            buf = dbuf[d][par]
            return [
                pltpu.async_copy(
                    stack[d].at[src_idx.at[c]],
                    buf.at[pl.ds(c * _CHUNK, _CHUNK)],
                    gsem,
                )
                for c in range(_NCHUNK)
            ]

        pending = {}
        for e in extract(0):
            e.wait()
        inflight = fire(0)
        enext = extract(1)
        for i in range(_NFIELD):
            f, d, par = _SCHED[i]
            nxt = None
            if i + 1 < _NFIELD:
                for e in enext:
                    e.wait()
                _, d1, par1 = _SCHED[i + 1]
                wb = pending.pop((d1, par1), None)
                if wb is not None:
                    wb.wait()
                nxt = fire(i + 1)
                if i + 2 < _NFIELD:
                    enext = extract(i + 2)
            for c in inflight:
                c.wait()
            pending[(d, par)] = pltpu.async_copy(
                dbuf[d][par], outs[f].at[pl.ds(base, _BPW)], wsem
            )
            inflight = nxt
        for wb in pending.values():
            wb.wait()

    return run


_RUN = _make_kernel()


def _static_col_idx():
    import numpy as np

    w = np.arange(_NW)[:, None, None, None]
    f = np.arange(_NFIELD)[None, :, None, None]
    c = np.arange(_NCHUNK)[None, None, :, None]
    m = np.arange(_CHUNK)[None, None, None, :]
    flat = (w * _BPW + c * _CHUNK + m) * _NFIELD + f
    return flat.reshape(_NW, _NFIELD * _NCHUNK, _CHUNK).astype("int32")


_SIDX = _static_col_idx()


@jax.jit
def kernel(x, W16, W32, W64):
    xoff = x + jnp.asarray(_OFFS, dtype=jnp.int32)[None, :]
    return _RUN(
        xoff.reshape(_BATCH * _NFIELD),
        jnp.asarray(_SIDX),
        W16.reshape(9 * _VOCAB, 16),
        W32.reshape(9 * _VOCAB, 32),
        W64.reshape(8 * _VOCAB, 64),
    )


# speedup vs baseline: 1.0001x; 1.0001x over previous
import functools

import jax
import jax.experimental.layout
import jax.numpy as jnp
from jax import lax
from jax.experimental import pallas as pl
from jax.experimental.pallas import tpu as pltpu
from jax.experimental.pallas import tpu_sc as plsc

_DIMS = ([16, 32, 64] * 8) + [16, 32]
_NFIELD = 26
_VOCAB = 100000
_BATCH = 16384
_NC = 2
_NS = 16
_NW = _NC * _NS
_BPW = _BATCH // _NW
_CHUNK = 128
_NCHUNK = _BPW // _CHUNK
_VLEN = 16

_GROUPS = {
    16: [f for f in range(_NFIELD) if _DIMS[f] == 16],
    32: [f for f in range(_NFIELD) if _DIMS[f] == 32],
    64: [f for f in range(_NFIELD) if _DIMS[f] == 64],
}
_SCHED = [
    (f, d, i % 2)
    for d in (16, 32, 64)
    for i, f in enumerate(_GROUPS[d])
]
_OFFS = [0] * _NFIELD
for _d, _fs in _GROUPS.items():
    for _i, _f in enumerate(_fs):
        _OFFS[_f] = _i * _VOCAB


def _make_kernel():
    mesh = plsc.VectorSubcoreMesh(core_axis_name="c", subcore_axis_name="s")
    out_type = tuple(
        jax.ShapeDtypeStruct((_BATCH, _DIMS[f]), jnp.float32)
        for f in range(_NFIELD)
    )

    @functools.partial(
        pl.kernel,
        mesh=mesh,
        out_type=out_type,
        compiler_params=pltpu.CompilerParams(use_tc_tiling_on_sc=False),
        scratch_types=[
            pltpu.VMEM((_NFIELD * _NCHUNK, _CHUNK), jnp.int32),
            pltpu.VMEM((3, _NCHUNK, _CHUNK), jnp.int32),
            pltpu.VMEM((_BPW, 16), jnp.float32),
            pltpu.VMEM((_BPW, 16), jnp.float32),
            pltpu.VMEM((_BPW, 32), jnp.float32),
            pltpu.VMEM((_BPW, 32), jnp.float32),
            pltpu.VMEM((_BPW, 64), jnp.float32),
            pltpu.VMEM((_BPW, 64), jnp.float32),
            pltpu.SemaphoreType.DMA,
            pltpu.SemaphoreType.DMA,
            pltpu.SemaphoreType.DMA,
        ],
    )
    def run(xflat_hbm, sidx_hbm, s16, s32, s64, *rest):
        outs = rest[:_NFIELD]
        (sidx_v, idxs, b16a, b16b, b32a, b32b, b64a, b64b,
         gsem, esem, wsem) = rest[_NFIELD:]
        idxb = tuple(idxs.at[k] for k in range(3))
        dbuf = {16: (b16a, b16b), 32: (b32a, b32b), 64: (b64a, b64b)}
        stack = {16: s16, 32: s32, 64: s64}

        wid = lax.axis_index("s") * _NC + lax.axis_index("c")
        base = wid * _BPW
        pltpu.sync_copy(sidx_hbm.at[wid], sidx_v)

        def extract(step):
            f, _, _ = _SCHED[step]
            dst = idxb[step % 3]
            return [
                pltpu.async_copy(
                    xflat_hbm.at[sidx_v.at[f * _NCHUNK + c]],
                    dst.at[c],
                    esem,
                )
                for c in range(_NCHUNK)
            ]

        def fire(step):
            _, d, par = _SCHED[step]
            src_idx = idxb[step % 3]
            buf = dbuf[d][par]
            return [
                pltpu.async_copy(
                    stack[d].at[src_idx.at[c]],
                    buf.at[pl.ds(c * _CHUNK, _CHUNK)],
                    gsem,
                )
                for c in range(_NCHUNK)
            ]

        pending = {}
        for e in extract(0):
            e.wait()
        inflight = fire(0)
        enext = extract(1)
        for i in range(_NFIELD):
            f, d, par = _SCHED[i]
            nxt = None
            if i + 1 < _NFIELD:
                for e in enext:
                    e.wait()
                _, d1, par1 = _SCHED[i + 1]
                wb = pending.pop((d1, par1), None)
                if wb is not None:
                    wb.wait()
                nxt = fire(i + 1)
                if i + 2 < _NFIELD:
                    enext = extract(i + 2)
            for c in inflight:
                c.wait()
            pending[(d, par)] = pltpu.async_copy(
                dbuf[d][par], outs[f].at[pl.ds(base, _BPW)], wsem
            )
            inflight = nxt
        for wb in pending.values():
            wb.wait()

    return run


_RUN = _make_kernel()


def _static_col_idx():
    import numpy as np

    w = np.arange(_NW)[:, None, None, None]
    f = np.arange(_NFIELD)[None, :, None, None]
    c = np.arange(_NCHUNK)[None, None, :, None]
    m = np.arange(_CHUNK)[None, None, None, :]
    flat = (w * _BPW + c * _CHUNK + m) * _NFIELD + f
    return flat.reshape(_NW, _NFIELD * _NCHUNK, _CHUNK).astype("int32")


_SIDX = _static_col_idx()

def _kernel_impl(x, W16, W32, W64):
    xoff = x + jnp.asarray(_OFFS, dtype=jnp.int32)[None, :]
    return _RUN(
        xoff.reshape(_BATCH * _NFIELD),
        jnp.asarray(_SIDX),
        W16.reshape(9 * _VOCAB, 16),
        W32.reshape(9 * _VOCAB, 32),
        W64.reshape(8 * _VOCAB, 64),
    )


_JITTED = {}


def kernel(x, W16, W32, W64):
    dev = None
    if isinstance(x, jax.Array) and not isinstance(x, jax.core.Tracer):
        dev = list(x.devices())[0]
    if dev not in _JITTED:
        if dev is None:
            _JITTED[dev] = jax.jit(_kernel_impl)
        else:
            sharding = jax.sharding.SingleDeviceSharding(dev)
            fmt = tuple(
                jax.experimental.layout.Format(
                    jax.experimental.layout.Layout((0, 1), tiling=()),
                    sharding,
                )
                for _ in range(_NFIELD)
            )
            _JITTED[dev] = jax.jit(_kernel_impl, out_shardings=fmt)
    return _JITTED[dev](x, W16, W32, W64)


# speedup vs baseline: 1.0182x; 1.0181x over previous
import functools

import jax
import jax.numpy as jnp
import numpy as np
from jax import lax
from jax.experimental import pallas as pl
from jax.experimental.pallas import tpu as pltpu
from jax.experimental.pallas import tpu_sc as plsc

_DIMS = ([16, 32, 64] * 8) + [16, 32]
_NFIELD = 26
_VOCAB = 100000
_BATCH = 16384
_NC = 2
_NS = 16
_NW = _NC * _NS
_BPW = _BATCH // _NW
_CHUNK = 128
_NCHUNK = _BPW // _CHUNK

_GROUPS = {
    16: [f for f in range(_NFIELD) if _DIMS[f] == 16],
    32: [f for f in range(_NFIELD) if _DIMS[f] == 32],
    64: [f for f in range(_NFIELD) if _DIMS[f] == 64],
}
_OFFS = [0] * _NFIELD
for _d, _fs in _GROUPS.items():
    for _i, _f in enumerate(_fs):
        _OFFS[_f] = _i * _VOCAB


def _static_col_idx(fields):
    w = np.arange(_NW)[:, None, None, None]
    f = np.asarray(fields)[None, :, None, None]
    c = np.arange(_NCHUNK)[None, None, :, None]
    m = np.arange(_CHUNK)[None, None, None, :]
    flat = (w * _BPW + c * _CHUNK + m) * _NFIELD + f
    return flat.reshape(_NW, len(fields) * _NCHUNK, _CHUNK).astype("int32")


_SIDX = {d: _static_col_idx(fs) for d, fs in _GROUPS.items()}


def _make_group_kernel(d, nf):
    mesh = plsc.VectorSubcoreMesh(core_axis_name="c", subcore_axis_name="s")
    out_type = tuple(
        jax.ShapeDtypeStruct((_BATCH, d), jnp.float32) for _ in range(nf)
    )

    @functools.partial(
        pl.kernel,
        mesh=mesh,
        out_type=out_type,
        compiler_params=pltpu.CompilerParams(use_tc_tiling_on_sc=False),
        scratch_types=[
            pltpu.VMEM((nf * _NCHUNK, _CHUNK), jnp.int32),
            pltpu.VMEM((3, _NCHUNK, _CHUNK), jnp.int32),
            pltpu.VMEM((_BPW, d), jnp.float32),
            pltpu.VMEM((_BPW, d), jnp.float32),
            pltpu.SemaphoreType.DMA,
            pltpu.SemaphoreType.DMA,
            pltpu.SemaphoreType.DMA,
        ],
    )
    def run(xflat_hbm, sidx_hbm, stack, *rest):
        outs = rest[:nf]
        sidx_v, idxs, bufa, bufb, gsem, esem, wsem = rest[nf:]
        idxb = tuple(idxs.at[k] for k in range(3))
        bufs = (bufa, bufb)

        wid = lax.axis_index("s") * _NC + lax.axis_index("c")
        base = wid * _BPW
        pltpu.sync_copy(sidx_hbm.at[wid], sidx_v)

        def extract(i):
            dst = idxb[i % 3]
            return [
                pltpu.async_copy(
                    xflat_hbm.at[sidx_v.at[i * _NCHUNK + c]],
                    dst.at[c],
                    esem,
                )
                for c in range(_NCHUNK)
            ]

        def fire(i):
            src_idx = idxb[i % 3]
            buf = bufs[i % 2]
            return [
                pltpu.async_copy(
                    stack.at[src_idx.at[c]],
                    buf.at[pl.ds(c * _CHUNK, _CHUNK)],
                    gsem,
                )
                for c in range(_NCHUNK)
            ]

        writeback = [None, None]
        for e in extract(0):
            e.wait()
        inflight = fire(0)
        enext = extract(1) if nf > 1 else None
        for i in range(nf):
            nxt = None
            if i + 1 < nf:
                for e in enext:
                    e.wait()
                if writeback[(i + 1) % 2] is not None:
                    writeback[(i + 1) % 2].wait()
                    writeback[(i + 1) % 2] = None
                nxt = fire(i + 1)
                if i + 2 < nf:
                    enext = extract(i + 2)
            for c in inflight:
                c.wait()
            writeback[i % 2] = pltpu.async_copy(
                bufs[i % 2], outs[i].at[pl.ds(base, _BPW)], wsem
            )
            inflight = nxt
        for wb in writeback:
            if wb is not None:
                wb.wait()

    return run


_RUNS = {d: _make_group_kernel(d, len(fs)) for d, fs in _GROUPS.items()}


@jax.jit
def kernel(x, W16, W32, W64):
    xoff = (x + jnp.asarray(_OFFS, dtype=jnp.int32)[None, :]).reshape(
        _BATCH * _NFIELD
    )
    stacks = {
        16: W16.reshape(9 * _VOCAB, 16),
        32: W32.reshape(9 * _VOCAB, 32),
        64: W64.reshape(8 * _VOCAB, 64),
    }
    results = [None] * _NFIELD
    for d in (64, 32, 16):
        fields = _GROUPS[d]
        outs = _RUNS[d](xoff, jnp.asarray(_SIDX[d]), stacks[d])
        for f, o in zip(fields, outs):
            results[f] = o
    return tuple(results)
